# Initial kernel scaffold; baseline (speedup 1.0000x reference)
import jax
import jax.numpy as jnp
from jax.experimental import pallas as pl


def kernel(x, edge_index, W1_0, b1_0, W2_0, b2_0, W1_1, b1_1, W2_1, b2_1, bn_g, bn_b, Wp, bp, pn_g, pn_b, prelu_w):
    raise NotImplementedError("write your pallas kernel here")


# speedup vs baseline: 5.6221x; 5.6221x over previous
import jax
import jax.numpy as jnp
from jax import lax
from jax.experimental import pallas as pl
from jax.experimental.pallas import tpu as pltpu
from jax.experimental.pallas import tpu_sc as plsc

N = 10000
E = 320000
D = 128
EPS = 1e-5

NC = 2
NS = 16
NW = NC * NS
EPW = E // NW
CHUNK = 128
NFULL = EPW // CHUNK
TAIL = EPW - NFULL * CHUNK
R0 = 624
RLAST = N - (NS - 1) * R0


def _segsum_body(src_hbm, dst_hbm, z_hbm, zeros_hbm, out_hbm,
                 idx_s, idx_d, rows, idx_st, idx_dt, rows_t, sem, agg):
    c = lax.axis_index("c")
    s = lax.axis_index("s")
    base = pl.multiple_of((c * NS + s) * EPW, 8)
    row0 = pl.multiple_of(s * R0, 8)

    @pl.when(s < NS - 1)
    def _():
        pltpu.sync_copy(zeros_hbm.at[pl.ds(0, R0)], agg.at[pl.ds(row0, R0)])

    @pl.when(s == NS - 1)
    def _():
        pltpu.sync_copy(zeros_hbm, agg.at[pl.ds((NS - 1) * R0, RLAST)])

    plsc.subcore_barrier()

    def body(i, carry):
        off = base + i * CHUNK
        pltpu.sync_copy(src_hbm.at[pl.ds(off, CHUNK)], idx_s)
        pltpu.sync_copy(dst_hbm.at[pl.ds(off, CHUNK)], idx_d)
        pltpu.async_copy(z_hbm.at[idx_s], rows, sem).wait()
        pltpu.sync_copy(rows, agg.at[idx_d], add=True)
        return carry

    lax.fori_loop(0, NFULL, body, 0)

    toff = base + NFULL * CHUNK
    pltpu.sync_copy(src_hbm.at[pl.ds(toff, TAIL)], idx_st)
    pltpu.sync_copy(dst_hbm.at[pl.ds(toff, TAIL)], idx_dt)
    pltpu.async_copy(z_hbm.at[idx_st], rows_t, sem).wait()
    pltpu.sync_copy(rows_t, agg.at[idx_dt], add=True)

    plsc.subcore_barrier()
    obase = pl.multiple_of(c * N + row0, 8)

    @pl.when(s < NS - 1)
    def _():
        pltpu.sync_copy(agg.at[pl.ds(row0, R0)], out_hbm.at[pl.ds(obase, R0)])

    @pl.when(s == NS - 1)
    def _():
        pltpu.sync_copy(agg.at[pl.ds((NS - 1) * R0, RLAST)],
                        out_hbm.at[pl.ds(c * N + (NS - 1) * R0, RLAST)])


def _segment_sum(z, src, dst, zeros):
    mesh = plsc.VectorSubcoreMesh(core_axis_name="c", subcore_axis_name="s")
    k = pl.kernel(
        _segsum_body,
        mesh=mesh,
        out_type=jax.ShapeDtypeStruct((2 * N, D), jnp.float32),
        scratch_types=[
            pltpu.VMEM((CHUNK,), jnp.int32),
            pltpu.VMEM((CHUNK,), jnp.int32),
            pltpu.VMEM((CHUNK, D), jnp.float32),
            pltpu.VMEM((TAIL,), jnp.int32),
            pltpu.VMEM((TAIL,), jnp.int32),
            pltpu.VMEM((TAIL, D), jnp.float32),
            pltpu.SemaphoreType.DMA,
            pltpu.VMEM_SHARED((N, D), jnp.float32),
        ],
    )
    return k(src, dst, z, zeros)


BM = 1000


def _mlp_body(x_ref, p0_ref, p1_ref, w1_ref, b1_ref, w2_ref, b2_ref, o_ref):
    h = x_ref[...] + p0_ref[...] + p1_ref[...]
    h = jnp.dot(h, w1_ref[...], preferred_element_type=jnp.float32) + b1_ref[...]
    h = jnp.maximum(h, 0.0)
    h = jnp.dot(h, w2_ref[...], preferred_element_type=jnp.float32) + b2_ref[...]
    o_ref[...] = jnp.maximum(h, 0.0)


def _gin_mlp(x, parts, w1, b1, w2, b2):
    nb = N // BM
    return pl.pallas_call(
        _mlp_body,
        grid=(nb,),
        in_specs=[
            pl.BlockSpec((BM, D), lambda i: (i, 0)),
            pl.BlockSpec((BM, D), lambda i: (i, 0)),
            pl.BlockSpec((BM, D), lambda i, nb=nb: (i + nb, 0)),
            pl.BlockSpec((D, D), lambda i: (0, 0)),
            pl.BlockSpec((1, D), lambda i: (0, 0)),
            pl.BlockSpec((D, D), lambda i: (0, 0)),
            pl.BlockSpec((1, D), lambda i: (0, 0)),
        ],
        out_specs=pl.BlockSpec((BM, D), lambda i: (i, 0)),
        out_shape=jax.ShapeDtypeStruct((N, D), jnp.float32),
    )(x, parts, parts, w1, b1.reshape(1, D), w2, b2.reshape(1, D))


def _final_body(z2_ref, wp_ref, bp_ref, bng_ref, bnb_ref, png_ref, pnb_ref,
                pw_ref, z_ref, p_ref):
    z2 = z2_ref[...]
    m = jnp.mean(z2, axis=0, keepdims=True)
    v = jnp.mean((z2 - m) ** 2, axis=0, keepdims=True)
    z = (z2 - m) / jnp.sqrt(v + EPS) * bng_ref[...] + bnb_ref[...]
    z_ref[...] = z
    pp = jnp.dot(z, wp_ref[...], preferred_element_type=jnp.float32) + bp_ref[...]
    m2 = jnp.mean(pp, axis=0, keepdims=True)
    v2 = jnp.mean((pp - m2) ** 2, axis=0, keepdims=True)
    p = (pp - m2) / jnp.sqrt(v2 + EPS) * png_ref[...] + pnb_ref[...]
    p_ref[...] = jnp.where(p >= 0.0, p, pw_ref[0, 0] * p)


def _final(z2, wp, bp, bn_g, bn_b, pn_g, pn_b, prelu_w):
    return pl.pallas_call(
        _final_body,
        out_shape=(
            jax.ShapeDtypeStruct((N, D), jnp.float32),
            jax.ShapeDtypeStruct((N, D), jnp.float32),
        ),
    )(z2, wp, bp.reshape(1, D), bn_g.reshape(1, D), bn_b.reshape(1, D),
      pn_g.reshape(1, D), pn_b.reshape(1, D), prelu_w.reshape(1, 1))


def kernel(x, edge_index, W1_0, b1_0, W2_0, b2_0, W1_1, b1_1, W2_1, b2_1,
           bn_g, bn_b, Wp, bp, pn_g, pn_b, prelu_w):
    src = edge_index[0]
    dst = edge_index[1]
    zeros = jnp.zeros((RLAST, D), jnp.float32)
    parts0 = _segment_sum(x, src, dst, zeros)
    z1 = _gin_mlp(x, parts0, W1_0, b1_0, W2_0, b2_0)
    parts1 = _segment_sum(z1, src, dst, zeros)
    z2 = _gin_mlp(z1, parts1, W1_1, b1_1, W2_1, b2_1)
    z, p = _final(z2, Wp, bp, bn_g, bn_b, pn_g, pn_b, prelu_w)
    return (z, p)
